# baseline (speedup 1.0000x reference)
import jax
import jax.numpy as jnp
from jax import lax
from jax.experimental import pallas as pl
from jax.experimental.pallas import tpu as pltpu
from jax.experimental.pallas import tpu_sc as plsc

NUM_CORES = 2
NUM_SUBCORES = 16
LANES = 16
NW = NUM_CORES * NUM_SUBCORES

BATCH = 16384
DIM = 32
BPW = BATCH // NW
CHUNKS = BPW // LANES


def _bpr_body(uid_hbm, pid_hbm, nid_hbm, uemb_hbm, iemb_hbm,
              outp_hbm, outn_hbm,
              uidx_v, pidx_v, nidx_v, urows_v, prows_v, nrows_v,
              outp_v, outn_v, sem_u, sem_p, sem_n):
    wid = lax.axis_index("s") * NUM_CORES + lax.axis_index("c")
    base = wid * BPW

    pltpu.sync_copy(uid_hbm.at[pl.ds(base, BPW)], uidx_v)
    pltpu.sync_copy(pid_hbm.at[pl.ds(base, BPW)], pidx_v)
    pltpu.sync_copy(nid_hbm.at[pl.ds(base, BPW)], nidx_v)

    cu = pltpu.async_copy(uemb_hbm.at[uidx_v], urows_v, sem_u)
    cp = pltpu.async_copy(iemb_hbm.at[pidx_v], prows_v, sem_p)
    cn = pltpu.async_copy(iemb_hbm.at[nidx_v], nrows_v, sem_n)
    cu.wait()
    cp.wait()
    cn.wait()

    def chunk(c, carry):
        rows = c * LANES + lax.iota(jnp.int32, LANES)
        accp = jnp.zeros((LANES,), jnp.float32)
        accn = jnp.zeros((LANES,), jnp.float32)
        for d in range(DIM):
            dv = jnp.full((LANES,), d, jnp.int32)
            u = plsc.load_gather(urows_v, [rows, dv])
            p = plsc.load_gather(prows_v, [rows, dv])
            n = plsc.load_gather(nrows_v, [rows, dv])
            accp = accp + u * p
            accn = accn + u * n
        outp_v[pl.ds(c * LANES, LANES)] = accp
        outn_v[pl.ds(c * LANES, LANES)] = accn
        return carry

    lax.fori_loop(0, CHUNKS, chunk, 0)

    pltpu.sync_copy(outp_v, outp_hbm.at[pl.ds(base, BPW)])
    pltpu.sync_copy(outn_v, outn_hbm.at[pl.ds(base, BPW)])


def kernel(user_ids, pos_item_ids, neg_item_ids, user_emb, item_emb):
    mesh = plsc.VectorSubcoreMesh(
        core_axis_name="c", subcore_axis_name="s",
        num_cores=NUM_CORES, num_subcores=NUM_SUBCORES)
    out_type = (jax.ShapeDtypeStruct((BATCH,), jnp.float32),
                jax.ShapeDtypeStruct((BATCH,), jnp.float32))
    scratch = [
        pltpu.VMEM((BPW,), jnp.int32),
        pltpu.VMEM((BPW,), jnp.int32),
        pltpu.VMEM((BPW,), jnp.int32),
        pltpu.VMEM((BPW, DIM), jnp.float32),
        pltpu.VMEM((BPW, DIM), jnp.float32),
        pltpu.VMEM((BPW, DIM), jnp.float32),
        pltpu.VMEM((BPW,), jnp.float32),
        pltpu.VMEM((BPW,), jnp.float32),
        pltpu.SemaphoreType.DMA,
        pltpu.SemaphoreType.DMA,
        pltpu.SemaphoreType.DMA,
    ]
    f = pl.kernel(_bpr_body, out_type=out_type, mesh=mesh,
                  scratch_types=scratch,
                  compiler_params=pltpu.CompilerParams(
                      needs_layout_passes=False,
                      use_tc_tiling_on_sc=False))
    return f(user_ids.astype(jnp.int32), pos_item_ids.astype(jnp.int32),
             neg_item_ids.astype(jnp.int32), user_emb, item_emb)
